# baseline (speedup 1.0000x reference)
import functools

import jax
import jax.numpy as jnp
from jax import lax
from jax.experimental import pallas as pl
from jax.experimental.pallas import tpu as pltpu
from jax.experimental.pallas import tpu_sc as plsc

H = 1024
EPS = 1e-12
L = 16
NC = 2
NS = 16
NW = NC * NS
HV = H // L


def _rsqrt(x):
    i = lax.bitcast_convert_type(x, jnp.int32)
    i = jnp.int32(0x5F3759DF) - lax.shift_right_arithmetic(i, 1)
    y = lax.bitcast_convert_type(i, jnp.float32)
    for _ in range(3):
        y = y * (1.5 - 0.5 * x * y * y)
    return y


def _sc_embed_ln(ids_flat, token_table, pos_table, gamma, beta, *,
                 nt, s_len, ch):
    rpw = nt // NW
    nch = rpw // ch

    mesh = plsc.VectorSubcoreMesh(
        core_axis_name="c", subcore_axis_name="s",
        num_cores=NC, num_subcores=NS)

    @functools.partial(
        pl.kernel,
        out_type=jax.ShapeDtypeStruct((nt, H), jnp.float32),
        mesh=mesh,
        scratch_types=[
            pltpu.VMEM((rpw,), jnp.int32),
            pltpu.VMEM((H,), jnp.float32),
            pltpu.VMEM((H,), jnp.float32),
            pltpu.VMEM((ch, H), jnp.float32),
            pltpu.VMEM((ch, H), jnp.float32),
            pltpu.SemaphoreType.DMA,
        ],
    )
    def k(ids_hbm, tok_hbm, pos_hbm, gamma_hbm, beta_hbm, out_hbm,
          idx_v, gamma_v, beta_v, tok_b, pos_b, sem):
        wid = lax.axis_index("s") * NC + lax.axis_index("c")
        base = wid * rpw
        pos0 = lax.rem(base, s_len)

        pltpu.sync_copy(ids_hbm.at[pl.ds(base, rpw)], idx_v)
        pltpu.sync_copy(gamma_hbm, gamma_v)
        pltpu.sync_copy(beta_hbm, beta_v)

        @pl.loop(0, nch)
        def chunk_loop(c):
            row0 = c * ch
            pltpu.async_copy(
                tok_hbm.at[idx_v.at[pl.ds(row0, ch)]], tok_b, sem).wait()
            pltpu.sync_copy(pos_hbm.at[pl.ds(pos0 + row0, ch)], pos_b)

            @pl.loop(0, ch)
            def row_loop(r):
                def acc_body(j, carry):
                    s1, s2 = carry
                    v = tok_b[r, pl.ds(j * L, L)] + pos_b[r, pl.ds(j * L, L)]
                    tok_b[r, pl.ds(j * L, L)] = v
                    return s1 + v, s2 + v * v

                s1, s2 = lax.fori_loop(
                    0, HV, acc_body,
                    (jnp.zeros((L,), jnp.float32),
                     jnp.zeros((L,), jnp.float32)))
                t1 = s1[0]
                t2 = s2[0]
                for i in range(1, L):
                    t1 = t1 + s1[i]
                    t2 = t2 + s2[i]
                mean = t1 * (1.0 / H)
                var = t2 * (1.0 / H) - mean * mean
                rstd = _rsqrt(var + EPS)

                def norm_body(j, _):
                    v = tok_b[r, pl.ds(j * L, L)]
                    g = gamma_v[pl.ds(j * L, L)]
                    b = beta_v[pl.ds(j * L, L)]
                    tok_b[r, pl.ds(j * L, L)] = (v - mean) * rstd * g + b
                    return 0

                lax.fori_loop(0, HV, norm_body, 0)

            pltpu.sync_copy(tok_b, out_hbm.at[pl.ds(base + row0, ch)])

    return k(ids_flat, token_table, pos_table, gamma, beta)


def kernel(input_ids, token_table, pos_table, gamma, beta):
    b, s = input_ids.shape
    nt = b * s
    ids_flat = input_ids.reshape(nt).astype(jnp.int32)
    out = _sc_embed_ln(ids_flat, token_table, pos_table, gamma, beta,
                       nt=nt, s_len=s, ch=16)
    return out.reshape(b, s, H)
